# baseline (speedup 1.0000x reference)
import functools

import jax
import jax.numpy as jnp
from jax import lax
from jax.experimental import pallas as pl
from jax.experimental.pallas import tpu as pltpu
from jax.experimental.pallas import tpu_sc as plsc

_BLOCK_S = 512


def _add_pe_tc_kernel(x_ref, pe_ref, o_ref):
    o_ref[...] = x_ref[...] + pe_ref[...][None, :, :]


def _tc_kernel(inputs, pos_table):
    B, S, D = inputs.shape
    grid = (S // _BLOCK_S,)
    return pl.pallas_call(
        _add_pe_tc_kernel,
        grid=grid,
        in_specs=[
            pl.BlockSpec((B, _BLOCK_S, D), lambda i: (0, i, 0)),
            pl.BlockSpec((_BLOCK_S, D), lambda i: (i, 0)),
        ],
        out_specs=pl.BlockSpec((B, _BLOCK_S, D), lambda i: (0, i, 0)),
        out_shape=jax.ShapeDtypeStruct((B, S, D), inputs.dtype),
        compiler_params=pltpu.CompilerParams(
            dimension_semantics=("parallel",),
        ),
    )(inputs, pos_table)


_CHUNK_W = 32768
_UNROLL = 8


def _make_sc_kernel(NWORDS, TWORDS):
    info = plsc.get_sparse_core_info()
    NC, NS, L = info.num_cores, info.num_subcores, info.num_lanes
    NW = NC * NS
    words_per_w = NWORDS // NW
    n_chunks = words_per_w // _CHUNK_W
    mesh = plsc.VectorSubcoreMesh(core_axis_name="c", subcore_axis_name="s")

    @functools.partial(
        pl.kernel,
        mesh=mesh,
        out_type=jax.ShapeDtypeStruct((NWORDS,), jnp.float32),
        scratch_types=[
            pltpu.VMEM((_CHUNK_W,), jnp.float32),
            pltpu.VMEM((_CHUNK_W,), jnp.float32),
        ],
    )
    def k(x_hbm, t_hbm, o_hbm, bufx, buft):
        wid = lax.axis_index("s") * NC + lax.axis_index("c")
        base = wid * words_per_w
        tbase = lax.rem(base, TWORDS)

        def add_chunk(i, _):
            off = i * (L * _UNROLL)
            for u in range(_UNROLL):
                s = pl.ds(off + u * L, L)
                bufx[s] = bufx[s] + buft[s]
            return 0

        for c in range(n_chunks):
            o0 = base + c * _CHUNK_W
            t0 = tbase + c * _CHUNK_W
            pltpu.sync_copy(x_hbm.at[pl.ds(o0, _CHUNK_W)], bufx)
            pltpu.sync_copy(t_hbm.at[pl.ds(t0, _CHUNK_W)], buft)
            lax.fori_loop(0, _CHUNK_W // (L * _UNROLL), add_chunk, 0)
            pltpu.sync_copy(bufx, o_hbm.at[pl.ds(o0, _CHUNK_W)])

    return k


def kernel(inputs, pos_table):
    B, S, D = inputs.shape
    x = inputs.reshape(B * S * D)
    t = pos_table.reshape(S * D)
    out = _make_sc_kernel(B * S * D, S * D)(x, t)
    return out.reshape(B, S, D)


# speedup vs baseline: 1.2366x; 1.2366x over previous
import functools

import jax
import jax.numpy as jnp
from jax import lax
from jax.experimental import pallas as pl
from jax.experimental.pallas import tpu as pltpu
from jax.experimental.pallas import tpu_sc as plsc

_BLOCK_S = 512


def _add_pe_tc_kernel(x_ref, pe_ref, o_ref):
    o_ref[...] = x_ref[...] + pe_ref[...][None, :, :]


def _tc_kernel(inputs, pos_table):
    B, S, D = inputs.shape
    grid = (S // _BLOCK_S,)
    return pl.pallas_call(
        _add_pe_tc_kernel,
        grid=grid,
        in_specs=[
            pl.BlockSpec((B, _BLOCK_S, D), lambda i: (0, i, 0)),
            pl.BlockSpec((_BLOCK_S, D), lambda i: (i, 0)),
        ],
        out_specs=pl.BlockSpec((B, _BLOCK_S, D), lambda i: (0, i, 0)),
        out_shape=jax.ShapeDtypeStruct((B, S, D), inputs.dtype),
        compiler_params=pltpu.CompilerParams(
            dimension_semantics=("parallel",),
        ),
    )(inputs, pos_table)


_CHUNK_W = 16384
_NBUF = 3
_UNROLL = 8


def _make_sc_kernel(NWORDS, TWORDS):
    info = plsc.get_sparse_core_info()
    NC, NS, L = info.num_cores, info.num_subcores, info.num_lanes
    NW = NC * NS
    words_per_w = NWORDS // NW
    n_chunks = words_per_w // _CHUNK_W
    mesh = plsc.VectorSubcoreMesh(core_axis_name="c", subcore_axis_name="s")

    @functools.partial(
        pl.kernel,
        mesh=mesh,
        out_type=jax.ShapeDtypeStruct((NWORDS,), jnp.float32),
        scratch_types=(
            [pltpu.VMEM((_CHUNK_W,), jnp.float32) for _ in range(2 * _NBUF)]
            + [pltpu.SemaphoreType.DMA for _ in range(3 * _NBUF)]
        ),
    )
    def k(x_hbm, t_hbm, o_hbm, *scratch):
        bufx = scratch[:_NBUF]
        buft = scratch[_NBUF:2 * _NBUF]
        semx = scratch[2 * _NBUF:3 * _NBUF]
        semt = scratch[3 * _NBUF:4 * _NBUF]
        semo = scratch[4 * _NBUF:5 * _NBUF]
        wid = lax.axis_index("s") * NC + lax.axis_index("c")
        base = wid * words_per_w
        tbase = lax.rem(base, TWORDS)

        fills = [None] * n_chunks
        stores = [None] * n_chunks

        def start_fill(c):
            b = c % _NBUF
            fx = pltpu.async_copy(
                x_hbm.at[pl.ds(base + c * _CHUNK_W, _CHUNK_W)], bufx[b], semx[b])
            ft = pltpu.async_copy(
                t_hbm.at[pl.ds(tbase + c * _CHUNK_W, _CHUNK_W)], buft[b], semt[b])
            fills[c] = (fx, ft)

        def make_add(b):
            def add_chunk(i, _):
                off = i * (L * _UNROLL)
                for u in range(_UNROLL):
                    s = pl.ds(off + u * L, L)
                    bufx[b][s] = bufx[b][s] + buft[b][s]
                return 0
            return add_chunk

        for c in range(min(_NBUF - 1, n_chunks)):
            start_fill(c)
        for c in range(n_chunks):
            b = c % _NBUF
            fills[c][0].wait()
            fills[c][1].wait()
            lax.fori_loop(0, _CHUNK_W // (L * _UNROLL), make_add(b), 0)
            stores[c] = pltpu.async_copy(
                bufx[b], o_hbm.at[pl.ds(base + c * _CHUNK_W, _CHUNK_W)], semo[b])
            nc = c + _NBUF - 1
            if nc < n_chunks:
                if nc - _NBUF >= 0:
                    stores[nc - _NBUF].wait()
                start_fill(nc)
        for c in range(max(0, n_chunks - _NBUF), n_chunks):
            stores[c].wait()

    return k


def kernel(inputs, pos_table):
    B, S, D = inputs.shape
    x = inputs.reshape(B * S * D)
    t = pos_table.reshape(S * D)
    out = _make_sc_kernel(B * S * D, S * D)(x, t)
    return out.reshape(B, S, D)
